# Initial kernel scaffold; baseline (speedup 1.0000x reference)
import jax
import jax.numpy as jnp
from jax.experimental import pallas as pl


def kernel(s, v, p, edge_index_local, d_local, a_local, r_local, e_local, edge_index_global, d_global, a_global, r_global, e_global, batch, params):
    raise NotImplementedError("write your pallas kernel here")


# speedup vs baseline: 8.9127x; 8.9127x over previous
import functools
import math

import jax
import jax.numpy as jnp
from jax import lax
from jax.experimental import pallas as pl
from jax.experimental.pallas import tpu as pltpu
from jax.experimental.pallas import tpu_sc as plsc

N = 50000
E = 800000
SDIM = 64
VDIM = 16
EDIM = 16
NL = 5
CUTOFF = 5.0

NB_NODE = 5000
EB_EDGE = 3200

_WGT = lambda shp: pl.BlockSpec(shp, lambda i: (0,) * len(shp))


def _ln_proj_body(s_ref, v_ref, g_ref, b_ref, vg_ref, wd_ref, ws_ref,
                  sln_ref, vln_ref, ad_ref, as_ref):
    s = s_ref[...]
    mu = jnp.mean(s, axis=-1, keepdims=True)
    xc = s - mu
    var = jnp.mean(xc * xc, axis=-1, keepdims=True)
    sln = xc * jax.lax.rsqrt(var + 1e-6) * g_ref[...] + b_ref[...]
    v = v_ref[...]
    vn2 = jnp.sum(v * v, axis=-1, keepdims=True) * (1.0 / VDIM)
    vln = v * jax.lax.rsqrt(vn2 + 1e-6) * vg_ref[...]
    sln_ref[...] = sln
    vln_ref[...] = vln
    ad_ref[...] = jnp.dot(sln, wd_ref[...], preferred_element_type=jnp.float32)
    as_ref[...] = jnp.dot(sln, ws_ref[...], preferred_element_type=jnp.float32)


def _ln_proj(s, v48, g, b, vg48, wdst, wsrc):
    nb = pl.BlockSpec((NB_NODE, SDIM), lambda i: (i, 0))
    vb = pl.BlockSpec((NB_NODE, 3 * VDIM), lambda i: (i, 0))
    return pl.pallas_call(
        _ln_proj_body,
        grid=(N // NB_NODE,),
        in_specs=[nb, vb, _WGT((1, SDIM)), _WGT((1, SDIM)), _WGT((1, 3 * VDIM)),
                  _WGT((SDIM, SDIM)), _WGT((SDIM, SDIM))],
        out_specs=[nb, vb, nb, nb],
        out_shape=[
            jax.ShapeDtypeStruct((N, SDIM), jnp.float32),
            jax.ShapeDtypeStruct((N, 3 * VDIM), jnp.float32),
            jax.ShapeDtypeStruct((N, SDIM), jnp.float32),
            jax.ShapeDtypeStruct((N, SDIM), jnp.float32),
        ],
    )(s, v48, g.reshape(1, -1), b.reshape(1, -1), vg48.reshape(1, -1), wdst, wsrc)


def _ln_out_body(s_ref, v_ref, g_ref, b_ref, vg_ref, sln_ref, vln_ref):
    s = s_ref[...]
    mu = jnp.mean(s, axis=-1, keepdims=True)
    xc = s - mu
    var = jnp.mean(xc * xc, axis=-1, keepdims=True)
    sln_ref[...] = xc * jax.lax.rsqrt(var + 1e-6) * g_ref[...] + b_ref[...]
    v = v_ref[...]
    vn2 = jnp.sum(v * v, axis=-1, keepdims=True) * (1.0 / VDIM)
    vln_ref[...] = v * jax.lax.rsqrt(vn2 + 1e-6) * vg_ref[...]


def _ln_out(s, v48, g, b, vg48):
    nb = pl.BlockSpec((NB_NODE, SDIM), lambda i: (i, 0))
    vb = pl.BlockSpec((NB_NODE, 3 * VDIM), lambda i: (i, 0))
    return pl.pallas_call(
        _ln_out_body,
        grid=(N // NB_NODE,),
        in_specs=[nb, vb, _WGT((1, SDIM)), _WGT((1, SDIM)), _WGT((1, 3 * VDIM))],
        out_specs=[nb, vb],
        out_shape=[
            jax.ShapeDtypeStruct((N, SDIM), jnp.float32),
            jax.ShapeDtypeStruct((N, 3 * VDIM), jnp.float32),
        ],
    )(s, v48, g.reshape(1, -1), b.reshape(1, -1), vg48.reshape(1, -1))


def _edge_body(has_v, gs_ref, gv_ref, dae_ref, r_ref, w1c_ref, b1_ref,
               w2_ref, b2_ref, ms_ref, vm_ref):
    dae = dae_ref[...]
    x = gs_ref[...] + jnp.dot(dae, w1c_ref[...], preferred_element_type=jnp.float32) + b1_ref[...]
    h = x * jax.nn.sigmoid(x)
    m = jnp.dot(h, w2_ref[...], preferred_element_type=jnp.float32) + b2_ref[...]
    d = dae[:, 0:1]
    w = 0.5 * (jnp.cos((math.pi / CUTOFF) * d) + 1.0) * (d < CUTOFF).astype(jnp.float32)
    ms_ref[...] = m[:, :SDIM] * w
    gr = m[:, SDIM:SDIM + VDIM] * w
    gv = m[:, SDIM + VDIM:] * w
    r = r_ref[...]
    parts = []
    for c in range(3):
        vc = gr * r[:, c:c + 1]
        if has_v:
            vc = vc + gv * gv_ref[:, c * VDIM:(c + 1) * VDIM]
        parts.append(vc)
    vm_ref[...] = jnp.concatenate(parts, axis=-1)


def _edge_mlp(gs, gv, dae, r3, w1c, b1, w2, b2, has_v):
    eb = pl.BlockSpec((EB_EDGE, SDIM), lambda i: (i, 0))
    vb = pl.BlockSpec((EB_EDGE, 3 * VDIM), lambda i: (i, 0))
    ins = [eb, vb, pl.BlockSpec((EB_EDGE, 18), lambda i: (i, 0)),
           pl.BlockSpec((EB_EDGE, 3), lambda i: (i, 0)),
           _WGT((18, SDIM)), _WGT((1, SDIM)),
           _WGT((SDIM, SDIM + 2 * VDIM)), _WGT((1, SDIM + 2 * VDIM))]
    return pl.pallas_call(
        functools.partial(_edge_body, has_v),
        grid=(E // EB_EDGE,),
        in_specs=ins,
        out_specs=[eb, vb],
        out_shape=[
            jax.ShapeDtypeStruct((E, SDIM), jnp.float32),
            jax.ShapeDtypeStruct((E, 3 * VDIM), jnp.float32),
        ],
    )(gs, gv, dae, r3, w1c, b1.reshape(1, -1), w2, b2.reshape(1, -1))


def _update_body(has_mlp, sln_ref, vln_ref, sagg_ref, vsum_ref, cnt_ref,
                 w1a_ref, w1b_ref, b1_ref, w2_ref, b2_ref, s_ref, v_ref):
    sagg = sagg_ref[...]
    s_new = sln_ref[...] + sagg
    inv = 1.0 / jnp.maximum(cnt_ref[...], 1.0)
    v_new = vln_ref[...] + vsum_ref[...] * inv
    if has_mlp:
        x = (jnp.dot(s_new, w1a_ref[...], preferred_element_type=jnp.float32)
             + jnp.dot(sagg, w1b_ref[...], preferred_element_type=jnp.float32)
             + b1_ref[...])
        h = x * jax.nn.sigmoid(x)
        u = jnp.dot(h, w2_ref[...], preferred_element_type=jnp.float32) + b2_ref[...]
        s_new = s_new + u[:, :SDIM]
        gate = jax.nn.sigmoid(u[:, SDIM:])
        v_new = v_new * jnp.concatenate([gate, gate, gate], axis=-1)
    s_ref[...] = s_new
    v_ref[...] = v_new


def _update(sln, vln, sagg, vsum, cnt, w1a, w1b, b1, w2, b2, has_mlp):
    nb = pl.BlockSpec((NB_NODE, SDIM), lambda i: (i, 0))
    vb = pl.BlockSpec((NB_NODE, 3 * VDIM), lambda i: (i, 0))
    cb = pl.BlockSpec((NB_NODE, 1), lambda i: (i, 0))
    return pl.pallas_call(
        functools.partial(_update_body, has_mlp),
        grid=(N // NB_NODE,),
        in_specs=[nb, vb, nb, vb, cb, _WGT((SDIM, SDIM)), _WGT((SDIM, SDIM)),
                  _WGT((1, SDIM)), _WGT((SDIM, SDIM + VDIM)), _WGT((1, SDIM + VDIM))],
        out_specs=[nb, vb],
        out_shape=[
            jax.ShapeDtypeStruct((N, SDIM), jnp.float32),
            jax.ShapeDtypeStruct((N, 3 * VDIM), jnp.float32),
        ],
    )(sln, vln, sagg, vsum, cnt, w1a, w1b, b1.reshape(1, -1), w2, b2.reshape(1, -1))


def _gather(ad, asrc, vln, src, dst, has_v):
    gs = ad[dst] + asrc[src]
    gv = vln[src] if has_v else jnp.zeros((E, 3 * VDIM), jnp.float32)
    return gs, gv


def _scatter_s(ms, dst):
    return jax.ops.segment_sum(ms, dst, num_segments=N)


def _scatter_v(vm, dst, with_cnt):
    vsum = jax.ops.segment_sum(vm, dst, num_segments=N)
    cnt = None
    if with_cnt:
        cnt = jax.ops.segment_sum(jnp.ones((E,), jnp.float32), dst, num_segments=N)
    return vsum, cnt


def kernel(s, v, p, edge_index_local, d_local, a_local, r_local, e_local,
           edge_index_global, d_global, a_global, r_global, e_global, batch, params):
    src = edge_index_local[0]
    dst = edge_index_local[1]
    v48 = v.reshape(N, 3 * VDIM)
    dae = jnp.concatenate([d_local[:, None], a_local[:, None], e_local], axis=-1)

    cnt = None
    for i in range(NL):
        lp = params["layers"][i]
        has_v = i > 0
        has_mlp = i < NL - 1
        vg48 = jnp.tile(lp["ln_vg"], 3)
        w1_dst = lp["eW1"][:SDIM]
        w1_src = lp["eW1"][SDIM:2 * SDIM]
        w1_c = lp["eW1"][2 * SDIM:]
        sln, vln, ad, asrc = _ln_proj(s, v48, lp["ln_g"], lp["ln_b"], vg48,
                                      w1_dst, w1_src)
        gs, gv = _gather(ad, asrc, vln, src, dst, has_v)
        ms, vm = _edge_mlp(gs, gv, dae, r_local, w1_c, lp["eb1"],
                           lp["eW2"], lp["eb2"], has_v)
        sagg = _scatter_s(ms, dst)
        vsum, cnt_new = _scatter_v(vm, dst, with_cnt=(i == 0))
        if cnt_new is not None:
            cnt = cnt_new.reshape(N, 1)
        s, v48 = _update(sln, vln, sagg, vsum, cnt,
                         lp["uW1"][:SDIM], lp["uW1"][SDIM:], lp["ub1"],
                         lp["uW2"], lp["ub2"], has_mlp)

    on = params["out_norm"]
    s, v48 = _ln_out(s, v48, on["g"], on["b"], jnp.tile(on["vg"], 3))
    return (s, v48.reshape(N, 3, VDIM))


# speedup vs baseline: 17.1097x; 1.9197x over previous
import functools
import math

import jax
import jax.numpy as jnp
from jax import lax
from jax.experimental import pallas as pl
from jax.experimental.pallas import tpu as pltpu
from jax.experimental.pallas import tpu_sc as plsc

N = 50000
E = 800000
SDIM = 64
VDIM = 16
EDIM = 16
NL = 5
CUTOFF = 5.0

NB_NODE = 5000
EB_EDGE = 3200

_WGT = lambda shp: pl.BlockSpec(shp, lambda i: (0,) * len(shp))


def _ln_proj_body(s_ref, v_ref, g_ref, b_ref, vg_ref, wd_ref, ws_ref,
                  sln_ref, vln_ref, td_ref, ts_ref):
    s = s_ref[...]
    mu = jnp.mean(s, axis=-1, keepdims=True)
    xc = s - mu
    var = jnp.mean(xc * xc, axis=-1, keepdims=True)
    sln = xc * jax.lax.rsqrt(var + 1e-6) * g_ref[...] + b_ref[...]
    v = v_ref[...]
    vn2 = jnp.sum(v * v, axis=-1, keepdims=True) * (1.0 / VDIM)
    vln = v * jax.lax.rsqrt(vn2 + 1e-6) * vg_ref[...]
    sln_ref[...] = sln
    vln_ref[...] = vln
    nb = s.shape[0]
    ad = jnp.dot(sln, wd_ref[...], preferred_element_type=jnp.float32)
    asr = jnp.dot(sln, ws_ref[...], preferred_element_type=jnp.float32)
    td_ref[...] = jnp.concatenate(
        [ad, jnp.zeros((nb, 128 - SDIM), jnp.float32)], axis=-1)
    ts_ref[...] = jnp.concatenate(
        [asr, vln, jnp.zeros((nb, 128 - SDIM - 3 * VDIM), jnp.float32)],
        axis=-1)


def _ln_proj(s, v48, g, b, vg48, wdst, wsrc):
    nb = pl.BlockSpec((NB_NODE, SDIM), lambda i: (i, 0))
    vb = pl.BlockSpec((NB_NODE, 3 * VDIM), lambda i: (i, 0))
    tb = pl.BlockSpec((NB_NODE, 128), lambda i: (i, 0))
    return pl.pallas_call(
        _ln_proj_body,
        grid=(N // NB_NODE,),
        in_specs=[nb, vb, _WGT((1, SDIM)), _WGT((1, SDIM)), _WGT((1, 3 * VDIM)),
                  _WGT((SDIM, SDIM)), _WGT((SDIM, SDIM))],
        out_specs=[nb, vb, tb, tb],
        out_shape=[
            jax.ShapeDtypeStruct((N, SDIM), jnp.float32),
            jax.ShapeDtypeStruct((N, 3 * VDIM), jnp.float32),
            jax.ShapeDtypeStruct((N, 128), jnp.float32),
            jax.ShapeDtypeStruct((N, 128), jnp.float32),
        ],
    )(s, v48, g.reshape(1, -1), b.reshape(1, -1), vg48.reshape(1, -1), wdst, wsrc)


def _ln_out_body(s_ref, v_ref, g_ref, b_ref, vg_ref, sln_ref, vln_ref):
    s = s_ref[...]
    mu = jnp.mean(s, axis=-1, keepdims=True)
    xc = s - mu
    var = jnp.mean(xc * xc, axis=-1, keepdims=True)
    sln_ref[...] = xc * jax.lax.rsqrt(var + 1e-6) * g_ref[...] + b_ref[...]
    v = v_ref[...]
    vn2 = jnp.sum(v * v, axis=-1, keepdims=True) * (1.0 / VDIM)
    vln_ref[...] = v * jax.lax.rsqrt(vn2 + 1e-6) * vg_ref[...]


def _ln_out(s, v48, g, b, vg48):
    nb = pl.BlockSpec((NB_NODE, SDIM), lambda i: (i, 0))
    vb = pl.BlockSpec((NB_NODE, 3 * VDIM), lambda i: (i, 0))
    return pl.pallas_call(
        _ln_out_body,
        grid=(N // NB_NODE,),
        in_specs=[nb, vb, _WGT((1, SDIM)), _WGT((1, SDIM)), _WGT((1, 3 * VDIM))],
        out_specs=[nb, vb],
        out_shape=[
            jax.ShapeDtypeStruct((N, SDIM), jnp.float32),
            jax.ShapeDtypeStruct((N, 3 * VDIM), jnp.float32),
        ],
    )(s, v48, g.reshape(1, -1), b.reshape(1, -1), vg48.reshape(1, -1))


def _edge_body(has_v, gd_ref, gs_ref, dae_ref, r_ref, w1c_ref, b1_ref,
               w2_ref, b2_ref, msg_ref):
    dae = dae_ref[...]
    gd = gd_ref[...]
    gs = gs_ref[...]
    x = (gd[:, :SDIM] + gs[:, :SDIM]
         + jnp.dot(dae, w1c_ref[...], preferred_element_type=jnp.float32)
         + b1_ref[...])
    h = x * jax.nn.sigmoid(x)
    m = jnp.dot(h, w2_ref[...], preferred_element_type=jnp.float32) + b2_ref[...]
    d = dae[:, 0:1]
    w = 0.5 * (jnp.cos((math.pi / CUTOFF) * d) + 1.0) * (d < CUTOFF).astype(jnp.float32)
    ms = m[:, :SDIM] * w
    gr = m[:, SDIM:SDIM + VDIM] * w
    gv = m[:, SDIM + VDIM:] * w
    r = r_ref[...]
    parts = [ms]
    for c in range(3):
        vc = gr * r[:, c:c + 1]
        if has_v:
            vc = vc + gv * gs[:, SDIM + c * VDIM:SDIM + (c + 1) * VDIM]
        parts.append(vc)
    ne = dae.shape[0]
    parts.append(jnp.ones((ne, 1), jnp.float32))
    parts.append(jnp.zeros((ne, 15), jnp.float32))
    msg_ref[...] = jnp.concatenate(parts, axis=-1)


def _edge_mlp(gd, gs, dae, r3, w1c, b1, w2, b2, has_v):
    ne = dae.shape[0]
    tb = pl.BlockSpec((EB_EDGE, 128), lambda i: (i, 0))
    ins = [tb, tb, pl.BlockSpec((EB_EDGE, 18), lambda i: (i, 0)),
           pl.BlockSpec((EB_EDGE, 3), lambda i: (i, 0)),
           _WGT((18, SDIM)), _WGT((1, SDIM)),
           _WGT((SDIM, SDIM + 2 * VDIM)), _WGT((1, SDIM + 2 * VDIM))]
    return pl.pallas_call(
        functools.partial(_edge_body, has_v),
        grid=(ne // EB_EDGE,),
        in_specs=ins,
        out_specs=tb,
        out_shape=jax.ShapeDtypeStruct((ne, 128), jnp.float32),
    )(gd, gs, dae, r3, w1c, b1.reshape(1, -1), w2, b2.reshape(1, -1))


def _update_body(has_mlp, sln_ref, vln_ref, agg_ref, w1a_ref, w1b_ref,
                 b1_ref, w2_ref, b2_ref, s_ref, v_ref):
    agg = agg_ref[...]
    sagg = agg[:, :SDIM]
    vsum = agg[:, SDIM:SDIM + 3 * VDIM]
    cnt = agg[:, SDIM + 3 * VDIM:SDIM + 3 * VDIM + 1]
    s_new = sln_ref[...] + sagg
    inv = 1.0 / jnp.maximum(cnt, 1.0)
    v_new = vln_ref[...] + vsum * inv
    if has_mlp:
        x = (jnp.dot(s_new, w1a_ref[...], preferred_element_type=jnp.float32)
             + jnp.dot(sagg, w1b_ref[...], preferred_element_type=jnp.float32)
             + b1_ref[...])
        h = x * jax.nn.sigmoid(x)
        u = jnp.dot(h, w2_ref[...], preferred_element_type=jnp.float32) + b2_ref[...]
        s_new = s_new + u[:, :SDIM]
        gate = jax.nn.sigmoid(u[:, SDIM:])
        v_new = v_new * jnp.concatenate([gate, gate, gate], axis=-1)
    s_ref[...] = s_new
    v_ref[...] = v_new


def _update(sln, vln, agg, w1a, w1b, b1, w2, b2, has_mlp):
    nb = pl.BlockSpec((NB_NODE, SDIM), lambda i: (i, 0))
    vb = pl.BlockSpec((NB_NODE, 3 * VDIM), lambda i: (i, 0))
    tb = pl.BlockSpec((NB_NODE, 128), lambda i: (i, 0))
    return pl.pallas_call(
        functools.partial(_update_body, has_mlp),
        grid=(N // NB_NODE,),
        in_specs=[nb, vb, tb, _WGT((SDIM, SDIM)), _WGT((SDIM, SDIM)),
                  _WGT((1, SDIM)), _WGT((SDIM, SDIM + VDIM)), _WGT((1, SDIM + VDIM))],
        out_specs=[nb, vb],
        out_shape=[
            jax.ShapeDtypeStruct((N, SDIM), jnp.float32),
            jax.ShapeDtypeStruct((N, 3 * VDIM), jnp.float32),
        ],
    )(sln, vln, agg, w1a, w1b, b1.reshape(1, -1), w2, b2.reshape(1, -1))


E_PAD = 819200
NWORK = 32
ROWS_PW = E_PAD // (NWORK * 128)
GCH = 2


def _sc_gather_call():
    mesh = plsc.VectorSubcoreMesh(core_axis_name="c", subcore_axis_name="s")
    outs = [jax.ShapeDtypeStruct((E_PAD, 128), jnp.float32),
            jax.ShapeDtypeStruct((E_PAD, 128), jnp.float32)]
    scratch = [
        pltpu.VMEM((GCH, 128), jnp.int32),
        pltpu.VMEM((GCH, 128), jnp.int32),
        pltpu.VMEM((GCH * 128, 128), jnp.float32),
        pltpu.VMEM((GCH * 128, 128), jnp.float32),
        pltpu.SemaphoreType.DMA,
        pltpu.SemaphoreType.DMA,
        pltpu.SemaphoreType.DMA,
    ]

    def body(td_hbm, ts_hbm, dst_hbm, src_hbm, gd_hbm, gs_hbm,
             idxd, idxs, bufD, bufS, semI, semG, semO):
        wid = lax.axis_index("s") * 2 + lax.axis_index("c")
        row0 = wid * ROWS_PW

        def chunk(i, carry):
            r = row0 + i * GCH
            e0 = r * 128
            ci1 = pltpu.async_copy(dst_hbm.at[pl.ds(r, GCH)], idxd, semI)
            ci2 = pltpu.async_copy(src_hbm.at[pl.ds(r, GCH)], idxs, semI)
            ci1.wait()
            ci2.wait()
            g = []
            for j in range(GCH):
                sl = pl.ds(j * 128, 128)
                g.append(pltpu.async_copy(td_hbm.at[idxd.at[j]], bufD.at[sl], semG))
                g.append(pltpu.async_copy(ts_hbm.at[idxs.at[j]], bufS.at[sl], semG))
            for c in g:
                c.wait()
            o = [pltpu.async_copy(bufD, gd_hbm.at[pl.ds(e0, GCH * 128)], semO),
                 pltpu.async_copy(bufS, gs_hbm.at[pl.ds(e0, GCH * 128)], semO)]
            for c in o:
                c.wait()
            return carry

        lax.fori_loop(0, ROWS_PW // GCH, chunk, 0)

    return pl.kernel(body, out_type=outs, mesh=mesh, scratch_types=scratch)


def _gather(td, ts, dst2d, src2d):
    return _sc_gather_call()(td, ts, dst2d, src2d)


def kernel(s, v, p, edge_index_local, d_local, a_local, r_local, e_local,
           edge_index_global, d_global, a_global, r_global, e_global, batch, params):
    src = edge_index_local[0]
    dst = edge_index_local[1]
    v48 = v.reshape(N, 3 * VDIM)
    npad = E_PAD - E
    pad_ids = (jnp.arange(npad, dtype=jnp.int32) * 997) % N
    dst2d = jnp.concatenate([dst, pad_ids]).reshape(E_PAD // 128, 128)
    src2d = jnp.concatenate([src, pad_ids]).reshape(E_PAD // 128, 128)
    dst_seg = jnp.concatenate([dst, jnp.full((npad,), N, jnp.int32)])
    dae = jnp.concatenate([d_local[:, None], a_local[:, None], e_local], axis=-1)
    dae = jnp.concatenate([dae, jnp.zeros((npad, 18), jnp.float32)], axis=0)
    r_pad = jnp.concatenate([r_local, jnp.zeros((npad, 3), jnp.float32)], axis=0)

    for i in range(NL):
        lp = params["layers"][i]
        has_v = i > 0
        has_mlp = i < NL - 1
        vg48 = jnp.tile(lp["ln_vg"], 3)
        w1_dst = lp["eW1"][:SDIM]
        w1_src = lp["eW1"][SDIM:2 * SDIM]
        w1_c = lp["eW1"][2 * SDIM:]
        sln, vln, td, ts = _ln_proj(s, v48, lp["ln_g"], lp["ln_b"], vg48,
                                    w1_dst, w1_src)
        gd, gs = _gather(td, ts, dst2d, src2d)
        msg = _edge_mlp(gd, gs, dae, r_pad, w1_c, lp["eb1"],
                        lp["eW2"], lp["eb2"], has_v)
        agg = jax.ops.segment_sum(msg, dst_seg, num_segments=N)
        s, v48 = _update(sln, vln, agg,
                         lp["uW1"][:SDIM], lp["uW1"][SDIM:], lp["ub1"],
                         lp["uW2"], lp["ub2"], has_mlp)

    on = params["out_norm"]
    s, v48 = _ln_out(s, v48, on["g"], on["b"], jnp.tile(on["vg"], 3))
    return (s, v48.reshape(N, 3, VDIM))
